# baseline (speedup 1.0000x reference)
import jax
import jax.numpy as jnp
from jax import lax
from jax.experimental import pallas as pl
from jax.experimental.pallas import tpu as pltpu

_STREAM_BLOCK = 512
_TAIL_BLOCK = 512

_DN_T = (((1,), (1,)), ((), ()))


def _gcn_allpass_kernel(xa_ref, a_ref, w1_ref, w2_ref, w3_ref, w4_ref,
                        b1_ref, b2_ref, b3_ref, b4_ref, out_ref,
                        a16, ma, mb, m0t, rr):
    nstep = pl.num_programs(0)
    i = pl.program_id(0)
    rb = a_ref.shape[0]
    n = a_ref.shape[1]
    dg = out_ref.shape[0]
    od = xa_ref.shape[0] // 4
    nb = dg // od

    f32 = jnp.float32

    @pl.when(i == 0)
    def _prep():
        w1 = w1_ref[...]
        w2 = w2_ref[...]
        w3 = w3_ref[...]
        w4 = w4_ref[...]
        p = jnp.dot(w4, jnp.dot(w3, jnp.dot(w2, w1)))
        q2 = jnp.dot(w4, jnp.dot(w3, w2))
        q3 = jnp.dot(w4, w3)

        for b in range(nb):
            m0t[:, b * od:(b + 1) * od] = lax.dot_general(
                xa_ref[b * od:(b + 1) * od, :], p,
                (((0,), (0,)), ((), ())), preferred_element_type=f32)
        r1 = lax.dot_general(b1_ref[...], q2, _DN_T,
                             preferred_element_type=f32)
        m0t[:, dg:] = jnp.broadcast_to(r1, (n, od))

        r2 = lax.dot_general(b2_ref[...], q3, _DN_T,
                             preferred_element_type=f32)
        r3 = lax.dot_general(b3_ref[...], w4, _DN_T,
                             preferred_element_type=f32)
        r4 = b4_ref[...]
        zg = jnp.zeros((1, dg), f32)
        rr[...] = jnp.concatenate(
            [jnp.zeros((1, dg + od), f32),
             jnp.concatenate([zg, r2], axis=1),
             jnp.concatenate([zg, r3], axis=1),
             jnp.concatenate([zg, r4], axis=1),
             jnp.zeros((4, dg + od), f32)], axis=0)

    a = a_ref[...]
    a16[pl.ds(i * rb, rb), :] = a.astype(jnp.bfloat16)
    ma[pl.ds(i * rb, rb), :] = jnp.dot(a, m0t[...],
                                       preferred_element_type=f32)

    @pl.when(i == nstep - 1)
    def _tail():
        tb = _TAIL_BLOCK
        nchunk = n // tb

        def one_pass(src, dst, p):
            mp = (src[...] + rr[p:p + 1, :]).astype(jnp.bfloat16)
            for j in range(nchunk):
                dst[j * tb:(j + 1) * tb, :] = jnp.dot(
                    a16[j * tb:(j + 1) * tb, :], mp,
                    preferred_element_type=f32)

        one_pass(ma, mb, 1)
        one_pass(mb, ma, 2)

        mp = (ma[...] + rr[3:4, :]).astype(jnp.bfloat16)
        for j in range(nchunk):
            res = jnp.dot(a16[j * tb:(j + 1) * tb, :], mp,
                          preferred_element_type=f32)
            comb = res[:, :dg] + jnp.concatenate([res[:, dg:]] * nb, axis=1)
            out_ref[:, j * tb:(j + 1) * tb] = comb.T


def kernel(x, adj, W1, b1, W2, b2, W3, b3, W4, b4):
    nb, in_dim, n = x.shape
    out_dim = W4.shape[0]
    f32 = jnp.float32
    dg = nb * out_dim
    w = dg + out_dim

    xa = x.reshape(nb * in_dim, n)
    rb = _STREAM_BLOCK
    full = lambda *s: pl.BlockSpec(s, lambda i: tuple(0 for _ in s))

    out = pl.pallas_call(
        _gcn_allpass_kernel,
        grid=(n // rb,),
        in_specs=[
            full(nb * in_dim, n),
            pl.BlockSpec((rb, n), lambda i: (i, 0)),
            full(*W1.shape), full(*W2.shape),
            full(*W3.shape), full(*W4.shape),
            full(1, b1.shape[0]), full(1, b2.shape[0]),
            full(1, b3.shape[0]), full(1, b4.shape[0]),
        ],
        out_specs=pl.BlockSpec((dg, n), lambda i: (0, 0)),
        out_shape=jax.ShapeDtypeStruct((dg, n), f32),
        scratch_shapes=[
            pltpu.VMEM((n, n), jnp.bfloat16),
            pltpu.VMEM((n, w), f32),
            pltpu.VMEM((n, w), f32),
            pltpu.VMEM((n, w), f32),
            pltpu.VMEM((8, w), f32),
        ],
        compiler_params=pltpu.CompilerParams(
            vmem_limit_bytes=100 * 1024 * 1024,
        ),
    )(xa, adj, W1, W2, W3, W4,
      b1.reshape(1, -1), b2.reshape(1, -1), b3.reshape(1, -1),
      b4.reshape(1, -1))

    return out.reshape(nb, out_dim, n)


# speedup vs baseline: 1.0910x; 1.0910x over previous
import jax
import jax.numpy as jnp
from jax import lax
from jax.experimental import pallas as pl
from jax.experimental.pallas import tpu as pltpu

_STREAM_BLOCK = 512
_TAIL_BLOCK = 256

_DN_T = (((1,), (1,)), ((), ()))


def _gcn_allpass_kernel(xa_ref, a_ref, w1_ref, w2_ref, w3_ref, w4_ref,
                        b1_ref, b2_ref, b3_ref, b4_ref, out_ref,
                        a16, ma, mb, m0t, rr):
    nstep = pl.num_programs(0)
    i = pl.program_id(0)
    rb = a_ref.shape[0]
    n = a_ref.shape[1]
    dg = out_ref.shape[0]
    od = xa_ref.shape[0] // 4
    nb = dg // od

    f32 = jnp.float32

    @pl.when(i == 0)
    def _prep():
        w1 = w1_ref[...]
        w2 = w2_ref[...]
        w3 = w3_ref[...]
        w4 = w4_ref[...]
        p = jnp.dot(w4, jnp.dot(w3, jnp.dot(w2, w1)))
        q2 = jnp.dot(w4, jnp.dot(w3, w2))
        q3 = jnp.dot(w4, w3)

        for b in range(nb):
            m0t[b * od:(b + 1) * od, :] = jnp.dot(
                p, xa_ref[b * od:(b + 1) * od, :],
                preferred_element_type=f32)
        r1c = lax.dot_general(q2, b1_ref[...], _DN_T,
                              preferred_element_type=f32)
        m0t[dg:, :] = jnp.broadcast_to(r1c, (od, n))

        r2 = lax.dot_general(b2_ref[...], q3, _DN_T,
                             preferred_element_type=f32)
        r3 = lax.dot_general(b3_ref[...], w4, _DN_T,
                             preferred_element_type=f32)
        r4 = b4_ref[...]
        zg = jnp.zeros((1, dg), f32)
        rr[...] = jnp.concatenate(
            [jnp.zeros((1, dg + od), f32),
             jnp.concatenate([zg, r2], axis=1),
             jnp.concatenate([zg, r3], axis=1),
             jnp.concatenate([zg, r4], axis=1),
             jnp.zeros((4, dg + od), f32)], axis=0)

    a = a_ref[...]
    a16[pl.ds(i * rb, rb), :] = a.astype(jnp.bfloat16)
    ma[pl.ds(i * rb, rb), :] = lax.dot_general(
        a, m0t[...], _DN_T, preferred_element_type=f32)

    @pl.when(i == nstep - 1)
    def _tail():
        tb = _TAIL_BLOCK
        nchunk = n // tb

        def one_pass(src, dst, p):
            mp = (src[...] + rr[p:p + 1, :]).astype(jnp.bfloat16)
            for j in range(nchunk):
                dst[j * tb:(j + 1) * tb, :] = jnp.dot(
                    a16[j * tb:(j + 1) * tb, :], mp,
                    preferred_element_type=f32)

        one_pass(ma, mb, 1)
        one_pass(mb, ma, 2)

        mp = (ma[...] + rr[3:4, :]).astype(jnp.bfloat16)
        for j in range(nchunk):
            res = jnp.dot(a16[j * tb:(j + 1) * tb, :], mp,
                          preferred_element_type=f32)
            comb = res[:, :dg] + jnp.concatenate([res[:, dg:]] * nb, axis=1)
            out_ref[:, j * tb:(j + 1) * tb] = comb.T


def kernel(x, adj, W1, b1, W2, b2, W3, b3, W4, b4):
    nb, in_dim, n = x.shape
    out_dim = W4.shape[0]
    f32 = jnp.float32
    dg = nb * out_dim
    w = dg + out_dim

    xa = x.reshape(nb * in_dim, n)
    rb = _STREAM_BLOCK
    full = lambda *s: pl.BlockSpec(s, lambda i: tuple(0 for _ in s))

    out = pl.pallas_call(
        _gcn_allpass_kernel,
        grid=(n // rb,),
        in_specs=[
            full(nb * in_dim, n),
            pl.BlockSpec((rb, n), lambda i: (i, 0)),
            full(*W1.shape), full(*W2.shape),
            full(*W3.shape), full(*W4.shape),
            full(1, b1.shape[0]), full(1, b2.shape[0]),
            full(1, b3.shape[0]), full(1, b4.shape[0]),
        ],
        out_specs=pl.BlockSpec((dg, n), lambda i: (0, 0)),
        out_shape=jax.ShapeDtypeStruct((dg, n), f32),
        scratch_shapes=[
            pltpu.VMEM((n, n), jnp.bfloat16),
            pltpu.VMEM((n, w), f32),
            pltpu.VMEM((n, w), f32),
            pltpu.VMEM((w, n), f32),
            pltpu.VMEM((8, w), f32),
        ],
        compiler_params=pltpu.CompilerParams(
            vmem_limit_bytes=100 * 1024 * 1024,
        ),
    )(xa, adj, W1, W2, W3, W4,
      b1.reshape(1, -1), b2.reshape(1, -1), b3.reshape(1, -1),
      b4.reshape(1, -1))

    return out.reshape(nb, out_dim, n)
